# baseline (device time: 49812 ns/iter reference)
import jax
import jax.numpy as jnp
from jax import lax
from jax.experimental import pallas as pl
from jax.experimental.pallas import tpu as pltpu

N_DEV = 4
B, Sq, Skv, Hq, Dh = 2, 256, 256, 16, 64
H_LOC = Hq // N_DEV
D_MODEL = 512
BLK = 64


def _ring_allreduce(partial):
    m, n = partial.shape

    def body(p_ref, out_ref, comm_ref, send_sems, recv_sems):
        my = lax.axis_index("i")
        left = lax.rem(my + N_DEV - 1, N_DEV)
        right = lax.rem(my + 1, N_DEV)

        barrier_sem = pltpu.get_barrier_semaphore()
        for nbr in (left, right):
            pl.semaphore_signal(
                barrier_sem, inc=1,
                device_id=(nbr,), device_id_type=pl.DeviceIdType.MESH,
            )
        pl.semaphore_wait(barrier_sem, 2)

        comm_ref[0] = p_ref[...]
        out_ref[...] = p_ref[...]

        for h in range(N_DEV - 1):
            rdma = pltpu.make_async_remote_copy(
                src_ref=comm_ref.at[h],
                dst_ref=comm_ref.at[h + 1],
                send_sem=send_sems.at[h],
                recv_sem=recv_sems.at[h],
                device_id=(right,),
                device_id_type=pl.DeviceIdType.MESH,
            )
            rdma.start()
            rdma.wait()
            out_ref[...] += comm_ref[h + 1]

    return pl.pallas_call(
        body,
        out_shape=jax.ShapeDtypeStruct((m, n), jnp.float32),
        in_specs=[pl.BlockSpec(memory_space=pltpu.VMEM)],
        out_specs=pl.BlockSpec(memory_space=pltpu.VMEM),
        scratch_shapes=[
            pltpu.VMEM((N_DEV, m, n), jnp.float32),
            pltpu.SemaphoreType.DMA((N_DEV - 1,)),
            pltpu.SemaphoreType.DMA((N_DEV - 1,)),
        ],
        compiler_params=pltpu.CompilerParams(collective_id=0),
    )(partial)


def kernel(x, Wq, K_ext, V_ext, Wo):
    i = lax.axis_index("i")

    Q = (x.reshape(B * Sq, D_MODEL) @ Wq).reshape(B, Sq, H_LOC, Dh)
    K = lax.dynamic_slice_in_dim(K_ext, i * H_LOC, H_LOC, axis=2)
    V = lax.dynamic_slice_in_dim(V_ext, i * H_LOC, H_LOC, axis=2)

    scores = jnp.einsum("bihd,bjhd->bhij", Q, K) * 0.125
    qb = jnp.arange(Sq) // BLK
    kb = jnp.arange(Skv) // BLK
    mask = kb[None, :] <= qb[:, None]
    scores = jnp.where(mask[None, None], scores, -1e9)
    w = jax.nn.softmax(scores, axis=-1)
    ctx = jnp.einsum("bhij,bjhd->bihd", w, V).reshape(B * Sq, H_LOC * Dh)

    partial = ctx @ Wo
    out = _ring_allreduce(partial)
    return out.reshape(B, Sq, D_MODEL)


# device time: 25866 ns/iter; 1.9258x vs baseline; 1.9258x over previous
import jax
import jax.numpy as jnp
from jax import lax
from jax.experimental import pallas as pl
from jax.experimental.pallas import tpu as pltpu

N_DEV = 4
B, Sq, Skv, Hq, Dh = 2, 256, 256, 16, 64
H_LOC = Hq // N_DEV
D_MODEL = 512
BLK = 64


def _ring_allreduce(partial):
    m, n = partial.shape
    mc = m // N_DEV

    def body(p_ref, out_ref, rs_buf, ag_src, ag_buf,
             rs_send, rs_recv, ag_send, ag_recv):
        my = lax.axis_index("i")

        barrier_sem = pltpu.get_barrier_semaphore()
        for d in range(1, N_DEV):
            tgt = lax.rem(my + d, N_DEV)
            pl.semaphore_signal(
                barrier_sem, inc=1,
                device_id=(tgt,), device_id_type=pl.DeviceIdType.MESH,
            )
        pl.semaphore_wait(barrier_sem, N_DEV - 1)

        rs_rdmas = []
        for d in range(1, N_DEV):
            tgt = lax.rem(my + d, N_DEV)
            rdma = pltpu.make_async_remote_copy(
                src_ref=p_ref.at[pl.ds(tgt * mc, mc), :],
                dst_ref=rs_buf.at[d - 1],
                send_sem=rs_send.at[d - 1],
                recv_sem=rs_recv.at[d - 1],
                device_id=(tgt,),
                device_id_type=pl.DeviceIdType.MESH,
            )
            rdma.start()
            rs_rdmas.append(rdma)

        for rdma in rs_rdmas:
            rdma.wait_recv()

        chunk = p_ref[pl.ds(my * mc, mc), :]
        for k in range(N_DEV - 1):
            chunk = chunk + rs_buf[k]
        ag_src[...] = chunk
        out_ref[pl.ds(my * mc, mc), :] = chunk

        ag_rdmas = []
        for d in range(1, N_DEV):
            tgt = lax.rem(my + d, N_DEV)
            rdma = pltpu.make_async_remote_copy(
                src_ref=ag_src,
                dst_ref=ag_buf.at[d - 1],
                send_sem=ag_send.at[d - 1],
                recv_sem=ag_recv.at[d - 1],
                device_id=(tgt,),
                device_id_type=pl.DeviceIdType.MESH,
            )
            rdma.start()
            ag_rdmas.append(rdma)

        for d in range(1, N_DEV):
            ag_rdmas[d - 1].wait_recv()
            src_dev = lax.rem(my + N_DEV - d, N_DEV)
            out_ref[pl.ds(src_dev * mc, mc), :] = ag_buf[d - 1]

        for rdma in rs_rdmas:
            rdma.wait_send()
        for rdma in ag_rdmas:
            rdma.wait_send()

    return pl.pallas_call(
        body,
        out_shape=jax.ShapeDtypeStruct((m, n), jnp.float32),
        in_specs=[pl.BlockSpec(memory_space=pltpu.VMEM)],
        out_specs=pl.BlockSpec(memory_space=pltpu.VMEM),
        scratch_shapes=[
            pltpu.VMEM((N_DEV - 1, mc, n), jnp.float32),
            pltpu.VMEM((mc, n), jnp.float32),
            pltpu.VMEM((N_DEV - 1, mc, n), jnp.float32),
            pltpu.SemaphoreType.DMA((N_DEV - 1,)),
            pltpu.SemaphoreType.DMA((N_DEV - 1,)),
            pltpu.SemaphoreType.DMA((N_DEV - 1,)),
            pltpu.SemaphoreType.DMA((N_DEV - 1,)),
        ],
        compiler_params=pltpu.CompilerParams(collective_id=0),
    )(partial)


def kernel(x, Wq, K_ext, V_ext, Wo):
    i = lax.axis_index("i")

    Q = (x.reshape(B * Sq, D_MODEL) @ Wq).reshape(B, Sq, H_LOC, Dh)
    K = lax.dynamic_slice_in_dim(K_ext, i * H_LOC, H_LOC, axis=2)
    V = lax.dynamic_slice_in_dim(V_ext, i * H_LOC, H_LOC, axis=2)

    scores = jnp.einsum("bihd,bjhd->bhij", Q, K) * 0.125
    qb = jnp.arange(Sq) // BLK
    kb = jnp.arange(Skv) // BLK
    mask = kb[None, :] <= qb[:, None]
    scores = jnp.where(mask[None, None], scores, -1e9)
    w = jax.nn.softmax(scores, axis=-1)
    ctx = jnp.einsum("bhij,bjhd->bihd", w, V).reshape(B * Sq, H_LOC * Dh)

    partial = ctx @ Wo
    out = _ring_allreduce(partial)
    return out.reshape(B, Sq, D_MODEL)
